# Initial kernel scaffold; baseline (speedup 1.0000x reference)
import jax
import jax.numpy as jnp
from jax.experimental import pallas as pl


def kernel(x, table):
    raise NotImplementedError("write your pallas kernel here")


# speedup vs baseline: 6.7758x; 6.7758x over previous
import functools

import jax
import jax.numpy as jnp
from jax import lax
from jax.experimental import pallas as pl
from jax.experimental.pallas import tpu as pltpu
from jax.experimental.pallas import tpu_sc as plsc

NUM_EMB = 100000
DIM = 32
BATCH = 16384
TIN = 200
B = BATCH * TIN
NW = 32
PW = B // NW
CHUNK = 1024
NCH = PW // CHUNK


_TROWS = NUM_EMB * DIM // 128
_TBLK = 1000


def _sin_body(t_ref, o_ref):
    x = t_ref[...]
    col = lax.broadcasted_iota(jnp.int32, x.shape, 1)
    o_ref[...] = jnp.where(col % DIM == 0, x, jnp.sin(x))


def _sin_transform(table):
    flat = table.reshape(_TROWS, 128)
    out = pl.pallas_call(
        _sin_body,
        out_shape=jax.ShapeDtypeStruct((_TROWS, 128), jnp.float32),
        grid=(_TROWS // _TBLK,),
        in_specs=[pl.BlockSpec((_TBLK, 128), lambda i: (i, 0))],
        out_specs=pl.BlockSpec((_TBLK, 128), lambda i: (i, 0)),
    )(flat)
    return out.reshape(NUM_EMB, DIM)


_MESH = plsc.VectorSubcoreMesh(core_axis_name="c", subcore_axis_name="s")


@functools.partial(
    pl.kernel,
    mesh=_MESH,
    out_type=jax.ShapeDtypeStruct((B, DIM), jnp.float32),
    compiler_params=pltpu.CompilerParams(use_tc_tiling_on_sc=False),
    scratch_types=[
        pltpu.VMEM((CHUNK,), jnp.int32),
        pltpu.VMEM((CHUNK, DIM), jnp.float32),
        pltpu.SemaphoreType.DMA,
    ],
)
def _sc_gather(table_hbm, idx_hbm, out_hbm, idx_v, rows_v, sem):
    wid = lax.axis_index("s") * 2 + lax.axis_index("c")
    wbase = wid * PW

    def body(i, carry):
        base = wbase + i * CHUNK
        pltpu.sync_copy(idx_hbm.at[pl.ds(base, CHUNK)], idx_v)
        pltpu.async_copy(table_hbm.at[idx_v], rows_v, sem).wait()
        pltpu.sync_copy(rows_v, out_hbm.at[pl.ds(base, CHUNK)])
        return carry

    lax.fori_loop(0, NCH, body, 0)


def kernel(x, table):
    table_t = _sin_transform(table)
    idx = x.reshape(B).astype(jnp.int32)
    out = _sc_gather(table_t, idx)
    return out.reshape(BATCH, TIN, DIM)


# speedup vs baseline: 7.1312x; 1.0524x over previous
import functools

import jax
import jax.numpy as jnp
from jax import lax
from jax.experimental import pallas as pl
from jax.experimental.pallas import tpu as pltpu
from jax.experimental.pallas import tpu_sc as plsc

NUM_EMB = 100000
DIM = 32
BATCH = 16384
TIN = 200
B = BATCH * TIN
NW = 32
PW = B // NW
CHUNK = 1600
NCH = PW // CHUNK
NB = 2


_TROWS = NUM_EMB * DIM // 128
_TBLK = 1000


def _sin_body(t_ref, o_ref):
    x = t_ref[...]
    col = lax.broadcasted_iota(jnp.int32, x.shape, 1)
    o_ref[...] = jnp.where(col % DIM == 0, x, jnp.sin(x))


def _sin_transform(table):
    flat = table.reshape(_TROWS, 128)
    out = pl.pallas_call(
        _sin_body,
        out_shape=jax.ShapeDtypeStruct((_TROWS, 128), jnp.float32),
        grid=(_TROWS // _TBLK,),
        in_specs=[pl.BlockSpec((_TBLK, 128), lambda i: (i, 0))],
        out_specs=pl.BlockSpec((_TBLK, 128), lambda i: (i, 0)),
    )(flat)
    return out.reshape(NUM_EMB, DIM)


_MESH = plsc.VectorSubcoreMesh(core_axis_name="c", subcore_axis_name="s")


@functools.partial(
    pl.kernel,
    mesh=_MESH,
    out_type=jax.ShapeDtypeStruct((B, DIM), jnp.float32),
    compiler_params=pltpu.CompilerParams(use_tc_tiling_on_sc=False),
    scratch_types=[
        pltpu.VMEM((NB, CHUNK), jnp.int32),
        pltpu.VMEM((NB, CHUNK, DIM), jnp.float32),
        pltpu.SemaphoreType.DMA((NB,)),
        pltpu.SemaphoreType.DMA((NB,)),
        pltpu.SemaphoreType.DMA((NB,)),
    ],
)
def _sc_gather(table_hbm, idx_hbm, out_hbm, idx_v, rows_v,
               idx_sem, g_sem, o_sem):
    wid = lax.axis_index("s") * 2 + lax.axis_index("c")
    wbase = wid * PW

    def idx_copy(i, b):
        return pltpu.make_async_copy(
            idx_hbm.at[pl.ds(wbase + i * CHUNK, CHUNK)],
            idx_v.at[b], idx_sem.at[b])

    def g_copy(b):
        return pltpu.make_async_copy(
            table_hbm.at[idx_v.at[b]], rows_v.at[b], g_sem.at[b])

    def o_copy(i, b):
        return pltpu.make_async_copy(
            rows_v.at[b], out_hbm.at[pl.ds(wbase + i * CHUNK, CHUNK)],
            o_sem.at[b])

    idx_copy(0, 0).start()
    idx_copy(0, 0).wait()
    g_copy(0).start()
    for b in range(1, NB):
        idx_copy(b, b).start()

    @pl.loop(0, NCH, step=NB)
    def _(g):
        for b in range(NB):
            i = g + b
            b1 = (b + 1) % NB
            g_copy(b).wait()
            o_copy(i, b).start()

            @pl.when(i + 1 < NCH)
            def _():
                idx_copy(i + 1, b1).wait()

                @pl.when(i >= 1)
                def _():
                    o_copy(i - 1, b1).wait()

                g_copy(b1).start()

            @pl.when(i + NB < NCH)
            def _():
                idx_copy(i + NB, b).start()

    for i in range(NCH - NB, NCH):
        o_copy(i, i % NB).wait()


def kernel(x, table):
    table_t = _sin_transform(table)
    idx = x.reshape(B).astype(jnp.int32)
    out = _sc_gather(table_t, idx)
    return out.reshape(BATCH, TIN, DIM)


# speedup vs baseline: 7.1527x; 1.0030x over previous
import functools

import jax
import jax.numpy as jnp
from jax import lax
from jax.experimental import pallas as pl
from jax.experimental.pallas import tpu as pltpu
from jax.experimental.pallas import tpu_sc as plsc

NUM_EMB = 100000
DIM = 32
BATCH = 16384
TIN = 200
TOTAL = BATCH * TIN
NW = 32
SPAN = TOTAL // NW
C = 1024
NCH = SPAN // C

_TROWS = NUM_EMB * DIM // 128
_TBLK = 1000


def _sin_body(t_ref, o_ref):
    x = t_ref[...]
    col = lax.broadcasted_iota(jnp.int32, x.shape, 1)
    o_ref[...] = jnp.where(col % DIM == 0, x, jnp.sin(x))


def _sin_transform(table):
    flat = table.reshape(_TROWS, 128)
    out = pl.pallas_call(
        _sin_body,
        out_shape=jax.ShapeDtypeStruct((_TROWS, 128), jnp.float32),
        grid=(_TROWS // _TBLK,),
        in_specs=[pl.BlockSpec((_TBLK, 128), lambda i: (i, 0))],
        out_specs=pl.BlockSpec((_TBLK, 128), lambda i: (i, 0)),
    )(flat)
    return out.reshape(NUM_EMB, DIM)


_MESH = plsc.VectorSubcoreMesh(core_axis_name="c", subcore_axis_name="s")


@functools.partial(
    pl.kernel,
    mesh=_MESH,
    out_type=jax.ShapeDtypeStruct((TOTAL, DIM), jnp.float32),
    compiler_params=pltpu.CompilerParams(use_tc_tiling_on_sc=False),
    scratch_types=[
        pltpu.VMEM((2, C), jnp.int32),
        pltpu.VMEM((2, C, DIM), jnp.float32),
        pltpu.SemaphoreType.DMA((2,)),
        pltpu.SemaphoreType.DMA((2,)),
        pltpu.SemaphoreType.DMA((2,)),
    ],
)
def _sc_gather(table_hbm, xf_hbm, out_hbm, idx_v, rows_v, i_sem, g_sem,
               o_sem):
    wid = lax.axis_index("s") * 2 + lax.axis_index("c")
    base = wid * SPAN

    def i_copy(k):
        s = k % 2
        return pltpu.make_async_copy(
            xf_hbm.at[pl.ds(base + k * C, C)], idx_v.at[s], i_sem.at[s])

    def g_copy(k):
        s = k % 2
        return pltpu.make_async_copy(
            table_hbm.at[idx_v.at[s]], rows_v.at[s], g_sem.at[s])

    def o_copy(k):
        s = k % 2
        return pltpu.make_async_copy(
            rows_v.at[s], out_hbm.at[pl.ds(base + k * C, C), :], o_sem.at[s])

    i_copy(0).start()
    i_copy(1).start()
    i_copy(0).wait()
    g_copy(0).start()

    @pl.loop(0, NCH)
    def _(k):
        @pl.when(k + 1 < NCH)
        def _():
            i_copy(k + 1).wait()

            @pl.when(k >= 1)
            def _():
                o_copy(k - 1).wait()

            g_copy(k + 1).start()

        g_copy(k).wait()

        @pl.when(k + 2 < NCH)
        def _():
            i_copy(k + 2).start()

        o_copy(k).start()

    o_copy(NCH - 2).wait()
    o_copy(NCH - 1).wait()


def kernel(x, table):
    table_t = _sin_transform(table)
    xf = x.astype(jnp.int32).reshape(TOTAL)
    out = _sc_gather(table_t, xf)
    return out.reshape(BATCH, TIN, DIM)
